# Initial kernel scaffold; baseline (speedup 1.0000x reference)
import jax
import jax.numpy as jnp
from jax.experimental import pallas as pl


def kernel(text, weight):
    raise NotImplementedError("write your pallas kernel here")


# speedup vs baseline: 1.2886x; 1.2886x over previous
import functools

import jax
import jax.numpy as jnp
from jax import lax
from jax.experimental import pallas as pl
from jax.experimental.pallas import tpu as pltpu
from jax.experimental.pallas import tpu_sc as plsc

NC = 2
NS = 16
NW = NC * NS

CHUNK = 128
GROUP_K = 10


@functools.lru_cache(maxsize=None)
def _build(n_total, vocab, embed):
    per_w = n_total // NW
    nch = per_w // CHUNK
    ngrp = nch // GROUP_K
    grp_rows = GROUP_K * CHUNK

    mesh = plsc.VectorSubcoreMesh(core_axis_name="c", subcore_axis_name="s")

    @functools.partial(
        pl.kernel,
        out_type=jax.ShapeDtypeStruct((NW, ngrp, grp_rows, embed), jnp.float32),
        mesh=mesh,
        scratch_types=[
            pltpu.VMEM((nch, CHUNK), jnp.int32),
            pltpu.VMEM((2, grp_rows, embed), jnp.float32),
            pltpu.SemaphoreType.DMA,
            pltpu.SemaphoreType.DMA,
        ],
        compiler_params=pltpu.CompilerParams(use_tc_tiling_on_sc=False),
    )
    def gather_kernel(idx_hbm, table_hbm, out_hbm, idx_v, rows_v, sem0, sem1):
        wid = lax.axis_index("s") * NC + lax.axis_index("c")
        pltpu.sync_copy(idx_hbm.at[wid], idx_v)

        sems = (sem0, sem1)

        def issue_group(g, bufset, sem):
            for k in range(GROUP_K):
                c = g * GROUP_K + k
                pltpu.async_copy(
                    table_hbm.at[idx_v.at[c]],
                    rows_v.at[bufset, pl.ds(k * CHUNK, CHUNK)],
                    sem,
                )

        def drain_group(bufset, sem):
            pltpu.make_async_copy(
                table_hbm.at[pl.ds(0, grp_rows)],
                rows_v.at[bufset],
                sem,
            ).wait()

        issue_group(0, 0, sems[0])

        def body(i, _):
            for half in range(2):
                g = 2 * i + half
                bufset = half
                nxt = g + 1

                @pl.when(nxt < ngrp)
                def _():
                    issue_group(nxt, 1 - bufset, sems[1 - bufset])

                drain_group(bufset, sems[bufset])
                pltpu.sync_copy(rows_v.at[bufset], out_hbm.at[wid, g])
            return 0

        lax.fori_loop(0, ngrp // 2, body, 0)

    return gather_kernel


def kernel(text, weight):
    b, l = text.shape
    vocab, embed = weight.shape
    n_total = b * l
    idx = text.reshape(NW, (n_total // NW) // CHUNK, CHUNK).astype(jnp.int32)
    out = _build(n_total, vocab, embed)(idx, weight)
    return out.reshape(b, l, embed)


# speedup vs baseline: 1.8081x; 1.4032x over previous
import functools

import jax
import jax.numpy as jnp
from jax import lax
from jax.experimental import pallas as pl
from jax.experimental.pallas import tpu as pltpu
from jax.experimental.pallas import tpu_sc as plsc

NC = 2
NS = 16
NW = NC * NS

B_PER_GROUP = 16
CH_PER_GROUP = 8


@functools.lru_cache(maxsize=None)
def _build(b, l, vocab, embed):
    n_total = b * l
    per_w = n_total // NW
    b_per_w = b // NW
    grp_rows = B_PER_GROUP * l
    ngrp = b_per_w // B_PER_GROUP
    chunk = grp_rows // CH_PER_GROUP
    nch = per_w // chunk

    mesh = plsc.VectorSubcoreMesh(core_axis_name="c", subcore_axis_name="s")

    @functools.partial(
        pl.kernel,
        out_type=jax.ShapeDtypeStruct((b, l, embed), jnp.float32),
        mesh=mesh,
        scratch_types=[
            pltpu.VMEM((nch, chunk), jnp.int32),
            pltpu.VMEM((2, grp_rows, embed), jnp.float32),
            pltpu.SemaphoreType.DMA,
            pltpu.SemaphoreType.DMA,
            pltpu.SemaphoreType.DMA,
            pltpu.SemaphoreType.DMA,
        ],
        compiler_params=pltpu.CompilerParams(use_tc_tiling_on_sc=False),
    )
    def gather_kernel(idx_hbm, table_hbm, out_hbm, idx_v, rows_v,
                      gsem0, gsem1, ssem0, ssem1):
        wid = lax.axis_index("s") * NC + lax.axis_index("c")
        b_base = wid * b_per_w
        pltpu.sync_copy(idx_hbm.at[wid], idx_v)

        gsems = (gsem0, gsem1)
        ssems = (ssem0, ssem1)

        def gather_descr(g, bufset, sem, k):
            c = g * CH_PER_GROUP + k
            return pltpu.make_async_copy(
                table_hbm.at[idx_v.at[c]],
                rows_v.at[bufset, pl.ds(k * chunk, chunk)],
                sem,
            )

        def store_descr(g, bufset, sem, j):
            return pltpu.make_async_copy(
                rows_v.at[bufset, pl.ds(j * l, l)],
                out_hbm.at[b_base + g * B_PER_GROUP + j],
                sem,
            )

        def issue_gathers(g, bufset):
            for k in range(CH_PER_GROUP):
                gather_descr(g, bufset, gsems[bufset], k).start()

        def drain_gathers(g, bufset):
            for k in range(CH_PER_GROUP):
                gather_descr(g, bufset, gsems[bufset], k).wait()

        def issue_stores(g, bufset):
            for j in range(B_PER_GROUP):
                store_descr(g, bufset, ssems[bufset], j).start()

        def drain_stores(g, bufset):
            for j in range(B_PER_GROUP):
                store_descr(g, bufset, ssems[bufset], j).wait()

        issue_gathers(0, 0)

        def body(i, _):
            for half in range(2):
                g = 2 * i + half
                bufset = half

                @pl.when(g + 1 < ngrp)
                def _():
                    @pl.when(g >= 1)
                    def _():
                        drain_stores(g - 1, 1 - bufset)

                    issue_gathers(g + 1, 1 - bufset)

                drain_gathers(g, bufset)
                issue_stores(g, bufset)
            return 0

        lax.fori_loop(0, ngrp // 2, body, 0)
        drain_stores(ngrp - 2, 0)
        drain_stores(ngrp - 1, 1)

    return gather_kernel


def kernel(text, weight):
    b, l = text.shape
    vocab, embed = weight.shape
    n_total = b * l
    per_w = n_total // NW
    chunk = (B_PER_GROUP * l) // CH_PER_GROUP
    idx = text.reshape(NW, per_w // chunk, chunk).astype(jnp.int32)
    return _build(b, l, vocab, embed)(idx, weight)
